# baseline (device time: 108028 ns/iter reference)
import jax
import jax.numpy as jnp
from jax import lax
from jax.experimental import pallas as pl
from jax.experimental.pallas import tpu as pltpu

N_DEV = 8
SQ = 2048
HQ = 8
DH = 128
D_MODEL = HQ * DH
SCALE = 0.08838834764831843
BLK = 64
RES = 4

QCH = SQ // N_DEV
CH = 272
PACK_ROWS = N_DEV * CH


CB = (0, 384, 768, D_MODEL)


def _hyper(m):
    return m ^ ((m >> 1) & 1)


def _bit(v, k):
    return (v >> k) & 1


def _cmap1(pos):
    return 4 * _bit(pos, 0) + 2 * _bit(pos, 2) + _bit(pos, 1)


def _cmap2(pos):
    return 4 * _bit(pos, 1) + 2 * _bit(pos, 0) + _bit(pos, 2)


def _rdma(sbuf, rbuf, ssem, rsem, p):
    return pltpu.make_async_remote_copy(
        src_ref=sbuf,
        dst_ref=rbuf,
        send_sem=ssem,
        recv_sem=rsem,
        device_id=(p,),
        device_id_type=pl.DeviceIdType.MESH,
    )


def _neighbor_barrier(h):
    barrier = pltpu.get_barrier_semaphore()
    for k in (0, 1, 2):
        p = _hyper(h ^ (1 << k))
        pl.semaphore_signal(
            barrier, inc=1, device_id=(p,), device_id_type=pl.DeviceIdType.MESH
        )
    pl.semaphore_wait(barrier, 3)


_RS_DIMS = ((2, 1, 0), (1, 0, 2), (0, 2, 1))
_RS_CMAPS = (None, _cmap1, _cmap2)


def _rs_body(
    o_ref, l_ref, out_ref, acc,
    sa0, ra0, sa1, ra1, sa2, ra2,
    sb0, rb0, sb1, rb1, sb2, rb2,
    sc0, rc0, sc1, rc1, sc2, rc2,
    ssa, rsa, ssb, rsb, ssc, rsc,
):
    f32, bf16 = jnp.float32, jnp.bfloat16
    my = lax.axis_index("i")
    h = _hyper(my)
    _neighbor_barrier(h)

    for g in range(N_DEV):
        acc[g * CH : g * CH + QCH, :] = o_ref[g].astype(f32)
        acc[g * CH + QCH : g * CH + QCH + 2, :] = l_ref[g].astype(f32)
        acc[g * CH + QCH + 2 : (g + 1) * CH, :] = jnp.zeros(
            (CH - QCH - 2, D_MODEL), f32
        )

    bufs = (
        ((sa0, ra0), (sa1, ra1), (sa2, ra2)),
        ((sb0, rb0), (sb1, rb1), (sb2, rb2)),
        ((sc0, rc0), (sc1, rc1), (sc2, rc2)),
    )
    sems = ((ssa, rsa), (ssb, rsb), (ssc, rsc))
    los = [h * 0, h * 0, h * 0]
    for r in range(3):
        half = 4 >> r
        rds = []
        keeps = []
        for w_i in range(3):
            k = _RS_DIMS[w_i][r]
            cmap = _RS_CMAPS[w_i]
            c0, c1 = CB[w_i], CB[w_i + 1]
            sbuf, rbuf = bufs[w_i][r]
            hb = _bit(h, k)
            keep = los[w_i] + hb * half
            send = los[w_i] + (1 - hb) * half
            if cmap is None:
                sbuf[:] = acc[pl.ds(send * CH, half * CH), c0:c1].astype(bf16)
            else:
                for j in range(half):
                    c = cmap(send + j)
                    sbuf[j * CH : (j + 1) * CH, :] = acc[
                        pl.ds(c * CH, CH), c0:c1
                    ].astype(bf16)
            rd = _rdma(
                sbuf, rbuf, sems[w_i][0].at[r], sems[w_i][1].at[r],
                _hyper(h ^ (1 << k)),
            )
            rd.start()
            rds.append(rd)
            keeps.append(keep)
            los[w_i] = keep

        for w_i in range(3):
            rds[w_i].wait()
            cmap = _RS_CMAPS[w_i]
            c0, c1 = CB[w_i], CB[w_i + 1]
            _, rbuf = bufs[w_i][r]
            if cmap is None:
                acc[pl.ds(keeps[w_i] * CH, half * CH), c0:c1] += rbuf[:].astype(
                    f32
                )
            else:
                for j in range(half):
                    c = cmap(keeps[w_i] + j)
                    acc[pl.ds(c * CH, CH), c0:c1] += rbuf[
                        j * CH : (j + 1) * CH, :
                    ].astype(f32)

    out_ref[:] = acc[pl.ds(h * CH, CH), :]


def _reduce_scatter(o3, l3):
    f32, bf16 = jnp.float32, jnp.bfloat16
    bufs = []
    for w_i in range(3):
        cw = CB[w_i + 1] - CB[w_i]
        for half in (4, 2, 1):
            bufs += [pltpu.VMEM((half * CH, cw), bf16)] * 2
    return pl.pallas_call(
        _rs_body,
        out_shape=jax.ShapeDtypeStruct((CH, D_MODEL), f32),
        in_specs=[
            pl.BlockSpec(memory_space=pltpu.VMEM),
            pl.BlockSpec(memory_space=pltpu.VMEM),
        ],
        out_specs=pl.BlockSpec(memory_space=pltpu.VMEM),
        scratch_shapes=[pltpu.VMEM((PACK_ROWS, D_MODEL), f32)]
        + bufs
        + [pltpu.SemaphoreType.DMA((3,))] * 6,
        compiler_params=pltpu.CompilerParams(collective_id=0),
    )(o3, l3)


_AG_DIMS = ((0, 1, 2), (1, 2, 0), (2, 0, 1))
_AG_CMAPS = (None, _cmap2, _cmap1)


def _ag_body(y_ref, out_ref, ssa, rsa, ssb, rsb, ssc, rsc):
    f32, bf16 = jnp.float32, jnp.bfloat16
    my = lax.axis_index("i")
    h = _hyper(my)
    _neighbor_barrier(h)

    out_ref[pl.ds(h * QCH, QCH), :] = y_ref[:].astype(bf16)

    sems = ((ssa, rsa), (ssb, rsb), (ssc, rsc))
    vs = [
        h,
        _bit(h, 1) + 2 * _bit(h, 2) + 4 * _bit(h, 0),
        _bit(h, 2) + 2 * _bit(h, 0) + 4 * _bit(h, 1),
    ]
    sidx = [0, 0, 0]
    for r in range(3):
        sz = 1 << r
        rds = []
        for w_i in range(3):
            k = _AG_DIMS[w_i][r]
            cmap = _AG_CMAPS[w_i]
            c0, c1 = CB[w_i], CB[w_i + 1]
            p = _hyper(h ^ (1 << k))
            if cmap is None:
                reg = out_ref.at[pl.ds(vs[w_i] * QCH, sz * QCH), c0:c1]
                rd = _rdma(
                    reg, reg, sems[w_i][0].at[r], sems[w_i][1].at[r], p
                )
                rd.start()
                rds.append(rd)
            else:
                for j in range(sz):
                    c = cmap(vs[w_i] + j)
                    reg = out_ref.at[pl.ds(c * QCH, QCH), c0:c1]
                    rd = _rdma(
                        reg, reg,
                        sems[w_i][0].at[sidx[w_i]], sems[w_i][1].at[sidx[w_i]],
                        p,
                    )
                    rd.start()
                    rds.append(rd)
                    sidx[w_i] += 1
        for rd in rds:
            rd.wait()
        for w_i in range(3):
            vs[w_i] = vs[w_i] & ~sz


def _all_gather(y_chunk):
    bf16 = jnp.bfloat16
    return pl.pallas_call(
        _ag_body,
        out_shape=jax.ShapeDtypeStruct((SQ, D_MODEL), bf16),
        in_specs=[pl.BlockSpec(memory_space=pltpu.VMEM)],
        out_specs=pl.BlockSpec(memory_space=pltpu.VMEM),
        scratch_shapes=[
            pltpu.SemaphoreType.DMA((3,)),
            pltpu.SemaphoreType.DMA((3,)),
            pltpu.SemaphoreType.DMA((7,)),
            pltpu.SemaphoreType.DMA((7,)),
            pltpu.SemaphoreType.DMA((7,)),
            pltpu.SemaphoreType.DMA((7,)),
        ],
        compiler_params=pltpu.CompilerParams(collective_id=1),
    )(y_chunk)


def _attn_body(xg_ref, wq_ref, kg_ref, vg_ref, o_ref, l_ref):
    f32, bf16 = jnp.float32, jnp.bfloat16
    xr = xg_ref[0]
    wq = wq_ref[...]
    q = jax.lax.dot(xr, wq, preferred_element_type=f32).astype(bf16)
    k = kg_ref[0, 0]
    s = jax.lax.dot_general(
        q, k, (((1,), (1,)), ((), ())), preferred_element_type=f32
    )
    w = jnp.exp(s.astype(bf16))
    l_ref[0, 0] = w.sum(axis=1, dtype=f32).reshape(4, 128)
    o = jax.lax.dot(w, vg_ref[0, 0], preferred_element_type=f32)
    o_ref[:, 0, :, :] = o.astype(bf16).reshape(8, BLK, DH)


def _attention(xg, Wqb, Kg, Vg):
    f32, bf16 = jnp.float32, jnp.bfloat16
    o_sd = jax.ShapeDtypeStruct((N_DEV, RES, BLK, D_MODEL), bf16)
    l_sd = jax.ShapeDtypeStruct((RES, HQ, 4, 128), f32)
    return pl.pallas_call(
        _attn_body,
        grid=(RES, HQ),
        in_specs=[
            pl.BlockSpec((1, 512, D_MODEL), lambda r, h: (r, 0, 0)),
            pl.BlockSpec((D_MODEL, DH), lambda r, h: (0, h)),
            pl.BlockSpec((1, 1, 512, DH), lambda r, h: (r, h, 0, 0)),
            pl.BlockSpec((1, 1, 512, DH), lambda r, h: (r, h, 0, 0)),
        ],
        out_shape=[o_sd, l_sd],
        out_specs=[
            pl.BlockSpec((N_DEV, 1, BLK, DH), lambda r, h: (0, r, 0, h)),
            pl.BlockSpec((1, 1, 4, 128), lambda r, h: (r, h, 0, 0)),
        ],
        compiler_params=pltpu.CompilerParams(
            dimension_semantics=("parallel", "parallel")
        ),
    )(xg, Wqb, Kg, Vg)


def kernel(x, Wq, K_ext, V_ext, Wo):
    f32 = jnp.float32
    bf16 = jnp.bfloat16

    xb = x[0].astype(bf16)
    K = K_ext[0].astype(bf16)
    V = V_ext[0].astype(bf16)

    xg = xb.reshape(8, RES, BLK, D_MODEL).transpose(1, 0, 2, 3).reshape(
        RES, 512, D_MODEL
    )
    Kg = K.reshape(8, RES, BLK, HQ, DH).transpose(1, 3, 0, 2, 4).reshape(
        RES, HQ, 512, DH
    )
    Vg = V.reshape(8, RES, BLK, HQ, DH).transpose(1, 3, 0, 2, 4).reshape(
        RES, HQ, 512, DH
    )

    o5, l_part = _attention(xg, (Wq * SCALE).astype(bf16), Kg, Vg)

    o3 = o5.reshape(N_DEV, QCH, D_MODEL)
    l3 = (
        l_part.reshape(RES, HQ, 8, BLK)
        .transpose(2, 0, 3, 1)
        .reshape(N_DEV, 2, D_MODEL)
        .astype(bf16)
    )

    chunk = _reduce_scatter(o3, l3)
    o_c = chunk[:QCH].reshape(QCH, HQ, DH)
    l_c = chunk[QCH : QCH + 2].reshape(QCH, HQ)
    ctx = (o_c / l_c[:, :, None]).astype(bf16).reshape(QCH, D_MODEL)
    y = jnp.dot(ctx, Wo.astype(bf16), preferred_element_type=f32)

    out = _all_gather(y)
    return out[None]


# device time: 107696 ns/iter; 1.0031x vs baseline; 1.0031x over previous
import jax
import jax.numpy as jnp
from jax import lax
from jax.experimental import pallas as pl
from jax.experimental.pallas import tpu as pltpu

N_DEV = 8
SQ = 2048
HQ = 8
DH = 128
D_MODEL = HQ * DH
SCALE = 0.08838834764831843
BLK = 64
RES = 4

QCH = SQ // N_DEV
CH = 272
PACK_ROWS = N_DEV * CH


CB = (0, 384, 768, D_MODEL)


def _hyper(m):
    return m ^ ((m >> 1) & 1)


def _bit(v, k):
    return (v >> k) & 1


def _cmap1(pos):
    return 4 * _bit(pos, 0) + 2 * _bit(pos, 2) + _bit(pos, 1)


def _cmap2(pos):
    return 4 * _bit(pos, 1) + 2 * _bit(pos, 0) + _bit(pos, 2)


def _rdma(sbuf, rbuf, ssem, rsem, p):
    return pltpu.make_async_remote_copy(
        src_ref=sbuf,
        dst_ref=rbuf,
        send_sem=ssem,
        recv_sem=rsem,
        device_id=(p,),
        device_id_type=pl.DeviceIdType.MESH,
    )


def _neighbor_barrier(h):
    barrier = pltpu.get_barrier_semaphore()
    for k in (0, 1, 2):
        p = _hyper(h ^ (1 << k))
        pl.semaphore_signal(
            barrier, inc=1, device_id=(p,), device_id_type=pl.DeviceIdType.MESH
        )
    pl.semaphore_wait(barrier, 3)


_RS_DIMS = ((2, 1, 0), (1, 0, 2), (0, 2, 1))
_RS_CMAPS = (None, _cmap1, _cmap2)


def _rs_body(
    o_ref, l_ref, out_ref, acc,
    sa0, ra0, sa1, ra1, sa2, ra2,
    sb0, rb0, sb1, rb1, sb2, rb2,
    sc0, rc0, sc1, rc1, sc2, rc2,
    ssa, rsa, ssb, rsb, ssc, rsc,
):
    f32, bf16 = jnp.float32, jnp.bfloat16
    my = lax.axis_index("i")
    h = _hyper(my)
    _neighbor_barrier(h)

    for g in range(N_DEV):
        acc[g * CH : g * CH + QCH, :] = o_ref[g].astype(f32)
        acc[g * CH + QCH : g * CH + QCH + 2, :] = l_ref[g].astype(f32)
        acc[g * CH + QCH + 2 : (g + 1) * CH, :] = jnp.zeros(
            (CH - QCH - 2, D_MODEL), f32
        )

    bufs = (
        ((sa0, ra0), (sa1, ra1), (sa2, ra2)),
        ((sb0, rb0), (sb1, rb1), (sb2, rb2)),
        ((sc0, rc0), (sc1, rc1), (sc2, rc2)),
    )
    sems = ((ssa, rsa), (ssb, rsb), (ssc, rsc))
    los = [h * 0, h * 0, h * 0]
    for r in range(3):
        half = 4 >> r
        rds = []
        keeps = []
        for w_i in range(3):
            k = _RS_DIMS[w_i][r]
            cmap = _RS_CMAPS[w_i]
            c0, c1 = CB[w_i], CB[w_i + 1]
            sbuf, rbuf = bufs[w_i][r]
            hb = _bit(h, k)
            keep = los[w_i] + hb * half
            send = los[w_i] + (1 - hb) * half
            if cmap is None:
                sbuf[:] = acc[pl.ds(send * CH, half * CH), c0:c1].astype(bf16)
            else:
                for j in range(half):
                    c = cmap(send + j)
                    sbuf[j * CH : (j + 1) * CH, :] = acc[
                        pl.ds(c * CH, CH), c0:c1
                    ].astype(bf16)
            rd = _rdma(
                sbuf, rbuf, sems[w_i][0].at[r], sems[w_i][1].at[r],
                _hyper(h ^ (1 << k)),
            )
            rd.start()
            rds.append(rd)
            keeps.append(keep)
            los[w_i] = keep

        for w_i in range(3):
            rds[w_i].wait()
            cmap = _RS_CMAPS[w_i]
            c0, c1 = CB[w_i], CB[w_i + 1]
            _, rbuf = bufs[w_i][r]
            if cmap is None:
                acc[pl.ds(keeps[w_i] * CH, half * CH), c0:c1] += rbuf[:].astype(
                    f32
                )
            else:
                for j in range(half):
                    c = cmap(keeps[w_i] + j)
                    acc[pl.ds(c * CH, CH), c0:c1] += rbuf[
                        j * CH : (j + 1) * CH, :
                    ].astype(f32)

    out_ref[:] = acc[pl.ds(h * CH, CH), :]


def _reduce_scatter(o3, l3):
    f32, bf16 = jnp.float32, jnp.bfloat16
    bufs = []
    for w_i in range(3):
        cw = CB[w_i + 1] - CB[w_i]
        for half in (4, 2, 1):
            bufs += [pltpu.VMEM((half * CH, cw), bf16)] * 2
    return pl.pallas_call(
        _rs_body,
        out_shape=jax.ShapeDtypeStruct((CH, D_MODEL), f32),
        in_specs=[
            pl.BlockSpec(memory_space=pltpu.VMEM),
            pl.BlockSpec(memory_space=pltpu.VMEM),
        ],
        out_specs=pl.BlockSpec(memory_space=pltpu.VMEM),
        scratch_shapes=[pltpu.VMEM((PACK_ROWS, D_MODEL), f32)]
        + bufs
        + [pltpu.SemaphoreType.DMA((3,))] * 6,
        compiler_params=pltpu.CompilerParams(collective_id=0),
    )(o3, l3)


_AG_DIMS = ((0, 1, 2), (1, 2, 0), (2, 0, 1))
_AG_CMAPS = (None, _cmap2, _cmap1)


def _ag_body(y_ref, out_ref, ssa, rsa, ssb, rsb, ssc, rsc):
    f32, bf16 = jnp.float32, jnp.bfloat16
    my = lax.axis_index("i")
    h = _hyper(my)
    _neighbor_barrier(h)

    out_ref[pl.ds(h * QCH, QCH), :] = y_ref[:].astype(bf16)

    sems = ((ssa, rsa), (ssb, rsb), (ssc, rsc))
    vs = [
        h,
        _bit(h, 1) + 2 * _bit(h, 2) + 4 * _bit(h, 0),
        _bit(h, 2) + 2 * _bit(h, 0) + 4 * _bit(h, 1),
    ]
    sidx = [0, 0, 0]
    for r in range(3):
        sz = 1 << r
        rds = []
        for w_i in range(3):
            k = _AG_DIMS[w_i][r]
            cmap = _AG_CMAPS[w_i]
            c0, c1 = CB[w_i], CB[w_i + 1]
            p = _hyper(h ^ (1 << k))
            if cmap is None:
                reg = out_ref.at[pl.ds(vs[w_i] * QCH, sz * QCH), c0:c1]
                rd = _rdma(
                    reg, reg, sems[w_i][0].at[r], sems[w_i][1].at[r], p
                )
                rd.start()
                rds.append(rd)
            else:
                for j in range(sz):
                    c = cmap(vs[w_i] + j)
                    reg = out_ref.at[pl.ds(c * QCH, QCH), c0:c1]
                    rd = _rdma(
                        reg, reg,
                        sems[w_i][0].at[sidx[w_i]], sems[w_i][1].at[sidx[w_i]],
                        p,
                    )
                    rd.start()
                    rds.append(rd)
                    sidx[w_i] += 1
        for rd in rds:
            rd.wait()
        for w_i in range(3):
            vs[w_i] = vs[w_i] & ~sz


def _all_gather(y_chunk):
    bf16 = jnp.bfloat16
    return pl.pallas_call(
        _ag_body,
        out_shape=jax.ShapeDtypeStruct((SQ, D_MODEL), bf16),
        in_specs=[pl.BlockSpec(memory_space=pltpu.VMEM)],
        out_specs=pl.BlockSpec(memory_space=pltpu.VMEM),
        scratch_shapes=[
            pltpu.SemaphoreType.DMA((3,)),
            pltpu.SemaphoreType.DMA((3,)),
            pltpu.SemaphoreType.DMA((7,)),
            pltpu.SemaphoreType.DMA((7,)),
            pltpu.SemaphoreType.DMA((7,)),
            pltpu.SemaphoreType.DMA((7,)),
        ],
        compiler_params=pltpu.CompilerParams(collective_id=1),
    )(y_chunk)


def _attn_body(xg_ref, wq_ref, kg_ref, vg_ref, o_ref, l_ref):
    f32, bf16 = jnp.float32, jnp.bfloat16
    xr = xg_ref[0]
    wq = wq_ref[...]
    q = jax.lax.dot(xr, wq, preferred_element_type=f32).astype(bf16)
    k = kg_ref[0, 0]
    s = jax.lax.dot_general(
        q, k, (((1,), (1,)), ((), ())), preferred_element_type=f32
    )
    w = jnp.exp(s)
    l_ref[0, 0] = w.sum(axis=1).reshape(4, 128)
    o = jax.lax.dot(
        w.astype(bf16), vg_ref[0, 0], preferred_element_type=f32
    )
    o_ref[:, 0, :, :] = o.astype(bf16).reshape(8, BLK, DH)


def _attention(xg, Wqb, Kg, Vg):
    f32, bf16 = jnp.float32, jnp.bfloat16
    o_sd = jax.ShapeDtypeStruct((N_DEV, RES, BLK, D_MODEL), bf16)
    l_sd = jax.ShapeDtypeStruct((RES, HQ, 4, 128), f32)
    return pl.pallas_call(
        _attn_body,
        grid=(RES, HQ),
        in_specs=[
            pl.BlockSpec((1, 512, D_MODEL), lambda r, h: (r, 0, 0)),
            pl.BlockSpec((D_MODEL, DH), lambda r, h: (0, h)),
            pl.BlockSpec((1, 1, 512, DH), lambda r, h: (r, h, 0, 0)),
            pl.BlockSpec((1, 1, 512, DH), lambda r, h: (r, h, 0, 0)),
        ],
        out_shape=[o_sd, l_sd],
        out_specs=[
            pl.BlockSpec((N_DEV, 1, BLK, DH), lambda r, h: (0, r, 0, h)),
            pl.BlockSpec((1, 1, 4, 128), lambda r, h: (r, h, 0, 0)),
        ],
        compiler_params=pltpu.CompilerParams(
            dimension_semantics=("parallel", "parallel")
        ),
    )(xg, Wqb, Kg, Vg)


def kernel(x, Wq, K_ext, V_ext, Wo):
    f32 = jnp.float32
    bf16 = jnp.bfloat16

    xb = x[0].astype(bf16)
    K = K_ext[0].astype(bf16)
    V = V_ext[0].astype(bf16)

    xg = xb.reshape(8, RES, BLK, D_MODEL).transpose(1, 0, 2, 3).reshape(
        RES, 512, D_MODEL
    )
    Kg = K.reshape(8, RES, BLK, HQ, DH).transpose(1, 3, 0, 2, 4).reshape(
        RES, HQ, 512, DH
    )
    Vg = V.reshape(8, RES, BLK, HQ, DH).transpose(1, 3, 0, 2, 4).reshape(
        RES, HQ, 512, DH
    )

    o5, l_part = _attention(xg, (Wq * SCALE).astype(bf16), Kg, Vg)

    o3 = o5.reshape(N_DEV, QCH, D_MODEL)
    l3 = (
        l_part.reshape(RES, HQ, 8, BLK)
        .transpose(2, 0, 3, 1)
        .reshape(N_DEV, 2, D_MODEL)
        .astype(bf16)
    )

    chunk = _reduce_scatter(o3, l3)
    o_c = chunk[:QCH].reshape(QCH, HQ, DH)
    l_c = chunk[QCH : QCH + 2].reshape(QCH, HQ)
    ctx = (o_c / l_c[:, :, None]).astype(bf16).reshape(QCH, D_MODEL)
    y = jnp.dot(ctx, Wo.astype(bf16), preferred_element_type=f32)

    out = _all_gather(y)
    return out[None]
